# Initial kernel scaffold; baseline (speedup 1.0000x reference)
import jax
import jax.numpy as jnp
from jax.experimental import pallas as pl


def kernel(W):
    raise NotImplementedError("write your pallas kernel here")


# speedup vs baseline: 6.5251x; 6.5251x over previous
import functools

import jax
import jax.numpy as jnp
from jax import lax
from jax.experimental import pallas as pl
from jax.experimental.pallas import tpu as pltpu
from jax.experimental.pallas import tpu_sc as plsc

B = 8
NA = 128
NB = 256
NSTEPS = NA + NB - 1
ROWP = NB + 1
MU_WORDS = 16 * (((NA + 1) * ROWP + 15) // 16)
L = 16
NVREG = NA // L
NEG = -1e20
LN2 = 0.6931471805599453


def _softlog13(s):
    hi = s >= 2.0
    y = jnp.where(hi, 0.5 * s, s)
    z = (y - 1.0) / (y + 1.0)
    z2 = z * z
    p = z2 * (1.0 / 9.0) + (1.0 / 7.0)
    p = p * z2 + (1.0 / 5.0)
    p = p * z2 + (1.0 / 3.0)
    p = p * z2 + 1.0
    return 2.0 * z * p + jnp.where(hi, LN2, 0.0)


def _lse3(a, b, c):
    m = jnp.maximum(jnp.maximum(a, b), c)
    s = jnp.exp(a - m) + jnp.exp(b - m) + jnp.exp(c - m)
    return m + _softlog13(s)


def _dtw_body(w_hbm, out_hbm, w_v, mu_v, out_v):
    wid = lax.axis_index("s") * 2 + lax.axis_index("c")

    @pl.when(wid < B)
    def _():
        pltpu.sync_copy(w_hbm.at[wid], w_v)

        lanes = lax.iota(jnp.int32, L)
        neg = jnp.full((L,), NEG, jnp.float32)

        def init(t, c):
            mu_v[pl.ds(t * L, L)] = neg
            return c
        lax.fori_loop(0, MU_WORDS // L, init, 0)
        mu_v[pl.ds(0, L)] = jnp.where(lanes == 0, 0.0, NEG)

        ivecs = [lanes + (v * L + 1) for v in range(NVREG)]
        c256 = [iv * (ROWP - 1) for iv in ivecs]
        c255 = [iv * (NB - 1) for iv in ivecs]

        def step(k, c):
            for v in range(NVREG):
                @pl.when(jnp.logical_and(v * L + 1 <= k - 1,
                                         v * L + L >= k - NB))
                def _():
                    fn = c256[v] + k
                    top = plsc.load_gather(mu_v, [fn - ROWP])
                    left = plsc.load_gather(mu_v, [fn - 1])
                    tl = plsc.load_gather(mu_v, [fn - (ROWP + 1)])
                    w = plsc.load_gather(w_v, [c255[v] + (k - ROWP)])
                    val = _lse3(top, left, tl) + w
                    msk = jnp.logical_and(ivecs[v] <= k - 1,
                                          ivecs[v] >= k - NB)
                    plsc.store_scatter(mu_v, [fn], val, mask=msk)
            return c
        lax.fori_loop(2, NA + NB + 1, step, 0)

        def derow(r, c):
            for g in range(NB // L):
                src = lanes + (r * ROWP + ROWP + 1 + g * L)
                out_v[pl.ds(r * NB + g * L, L)] = plsc.load_gather(mu_v, [src])
            return c
        lax.fori_loop(0, NA, derow, 0)
        pltpu.sync_copy(out_v, out_hbm.at[wid])


@jax.jit
def kernel(W):
    w_flat = W.reshape(B, NA * NB)
    mesh = plsc.VectorSubcoreMesh(core_axis_name="c", subcore_axis_name="s")
    out = pl.kernel(
        _dtw_body,
        mesh=mesh,
        compiler_params=pltpu.CompilerParams(needs_layout_passes=False),
        out_type=jax.ShapeDtypeStruct((B, NA * NB), jnp.float32),
        scratch_types=[
            pltpu.VMEM((NA * NB,), jnp.float32),
            pltpu.VMEM((MU_WORDS,), jnp.float32),
            pltpu.VMEM((NA * NB,), jnp.float32),
        ],
    )(w_flat)
    return out.reshape(B, NA, NB)


# speedup vs baseline: 9.1104x; 1.3962x over previous
import functools

import jax
import jax.numpy as jnp
from jax import lax
from jax.experimental import pallas as pl
from jax.experimental.pallas import tpu as pltpu
from jax.experimental.pallas import tpu_sc as plsc

B = 8
NA = 128
NB = 256
NSTEPS = NA + NB - 1
ROWP = NB + 1
MU_WORDS = 16 * (((NA + 1) * ROWP + 15) // 16)
L = 16
NVREG = NA // L
NEG = -1e20
LN2 = 0.6931471805599453


def _softlog13(s):
    hi = s >= 2.0
    y = jnp.where(hi, 0.5 * s, s)
    z = (y - 1.0) / (y + 1.0)
    z2 = z * z
    p = z2 * (1.0 / 9.0) + (1.0 / 7.0)
    p = p * z2 + (1.0 / 5.0)
    p = p * z2 + (1.0 / 3.0)
    p = p * z2 + 1.0
    return 2.0 * z * p + jnp.where(hi, LN2, 0.0)


def _lse3(a, b, c):
    m = jnp.maximum(jnp.maximum(a, b), c)
    s = jnp.exp(a - m) + jnp.exp(b - m) + jnp.exp(c - m)
    return m + _softlog13(s)


def _dtw_body(w_hbm, out_hbm, w_v, mu_v, out_v):
    wid = lax.axis_index("s") * 2 + lax.axis_index("c")

    @pl.when(wid < B)
    def _():
        pltpu.sync_copy(w_hbm.at[wid], w_v)

        lanes = lax.iota(jnp.int32, L)
        neg = jnp.full((L,), NEG, jnp.float32)

        for t in range(ROWP // L + 1):
            mu_v[pl.ds(t * L, L)] = neg
        mu_v[pl.ds(0, L)] = jnp.where(lanes == 0, 0.0, NEG)

        ivecs = [lanes + (v * L + 1) for v in range(NVREG)]
        c256 = [iv * (ROWP - 1) for iv in ivecs]
        c255 = [iv * (NB - 1) for iv in ivecs]
        for v in range(NVREG):
            plsc.store_scatter(mu_v, [ivecs[v] * ROWP], neg)

        def step(k, c):
            vals, fns, msks = [], [], []
            for v in range(NVREG):
                fn = c256[v] + k
                top = plsc.load_gather(mu_v, [fn - ROWP])
                left = plsc.load_gather(mu_v, [fn - 1])
                tl = plsc.load_gather(mu_v, [fn - (ROWP + 1)])
                w = plsc.load_gather(w_v, [c255[v] + (k - ROWP)])
                vals.append(_lse3(top, left, tl) + w)
                fns.append(fn)
                msks.append(jnp.logical_and(ivecs[v] <= k - 1,
                                            ivecs[v] >= k - NB))
            for v in range(NVREG):
                plsc.store_scatter(mu_v, [fns[v]], vals[v], mask=msks[v])
            return c
        lax.fori_loop(2, NA + NB + 1, step, 0)

        def derow(r, c):
            for g in range(NB // L):
                src = lanes + (r * ROWP + ROWP + 1 + g * L)
                out_v[pl.ds(r * NB + g * L, L)] = plsc.load_gather(mu_v, [src])
            return c
        lax.fori_loop(0, NA, derow, 0)
        pltpu.sync_copy(out_v, out_hbm.at[wid])


@jax.jit
def kernel(W):
    w_flat = W.reshape(B, NA * NB)
    mesh = plsc.VectorSubcoreMesh(core_axis_name="c", subcore_axis_name="s")
    out = pl.kernel(
        _dtw_body,
        mesh=mesh,
        compiler_params=pltpu.CompilerParams(needs_layout_passes=False),
        out_type=jax.ShapeDtypeStruct((B, NA * NB), jnp.float32),
        scratch_types=[
            pltpu.VMEM((NA * NB,), jnp.float32),
            pltpu.VMEM((MU_WORDS,), jnp.float32),
            pltpu.VMEM((NA * NB,), jnp.float32),
        ],
    )(w_flat)
    return out.reshape(B, NA, NB)


# speedup vs baseline: 21.5249x; 2.3627x over previous
import functools

import jax
import jax.numpy as jnp
from jax import lax
from jax.experimental import pallas as pl
from jax.experimental.pallas import tpu as pltpu
from jax.experimental.pallas import tpu_sc as plsc

B = 8
NA = 128
NB = 256
L = 16
NVREG = NA // L
NEG = -1e20
LN2 = 0.6931471805599453


def _softlog13(s):
    hi = s >= 2.0
    y = jnp.where(hi, 0.5 * s, s)
    z = (y - 1.0) / (y + 1.0)
    z2 = z * z
    p = z2 * (1.0 / 9.0) + (1.0 / 7.0)
    p = p * z2 + (1.0 / 5.0)
    p = p * z2 + (1.0 / 3.0)
    p = p * z2 + 1.0
    return 2.0 * z * p + jnp.where(hi, LN2, 0.0)


def _lse3(a, b, c):
    m = jnp.maximum(jnp.maximum(a, b), c)
    s = jnp.exp(a - m) + jnp.exp(b - m) + jnp.exp(c - m)
    return m + _softlog13(s)


def _dg(x, idx):
    return x.at[idx].get(mode="promise_in_bounds")


def _dtw_body(w_hbm, out_hbm, w_v, out_v):
    wid = lax.axis_index("s") * 2 + lax.axis_index("c")

    @pl.when(wid < B)
    def _():
        pltpu.sync_copy(w_hbm.at[wid], w_v)

        lanes = lax.iota(jnp.int32, L)
        neg = jnp.full((L,), NEG, jnp.float32)
        lane0 = lanes == 0
        sh_idx = jnp.maximum(lanes - 1, 0)
        hi_idx = jnp.full((L,), L - 1, jnp.int32)

        ivecs = [lanes + (v * L + 1) for v in range(NVREG)]
        c255 = [iv * (NB - 1) for iv in ivecs]

        d1 = [neg] * NVREG
        s1 = [neg] * NVREG
        s2 = [jnp.where(lane0, 0.0, NEG) if v == 0 else neg
              for v in range(NVREG)]

        def step(k, carry):
            d1 = carry[:NVREG]
            s1 = carry[NVREG:2 * NVREG]
            s2 = carry[2 * NVREG:]
            new, news = [], []
            for v in range(NVREG):
                wofs = c255[v] + (k - (NB + 1))
                w = plsc.load_gather(w_v, [wofs])
                msk = jnp.logical_and(ivecs[v] <= k - 1, ivecs[v] >= k - NB)
                val = jnp.where(msk, _lse3(d1[v], s1[v], s2[v]) + w, NEG)
                plsc.store_scatter(out_v, [wofs], val, mask=msk)
                carrier = neg if v == 0 else _dg(new[v - 1], hi_idx)
                news.append(jnp.where(lane0, carrier, _dg(val, sh_idx)))
                new.append(val)
            return tuple(new) + tuple(news) + tuple(s1)

        lax.fori_loop(2, NA + NB + 1, step,
                      tuple(d1) + tuple(s1) + tuple(s2))
        pltpu.sync_copy(out_v, out_hbm.at[wid])


@jax.jit
def kernel(W):
    w_flat = W.reshape(B, NA * NB)
    mesh = plsc.VectorSubcoreMesh(core_axis_name="c", subcore_axis_name="s")
    out = pl.kernel(
        _dtw_body,
        mesh=mesh,
        compiler_params=pltpu.CompilerParams(needs_layout_passes=False),
        out_type=jax.ShapeDtypeStruct((B, NA * NB), jnp.float32),
        scratch_types=[
            pltpu.VMEM((NA * NB,), jnp.float32),
            pltpu.VMEM((NA * NB,), jnp.float32),
        ],
    )(w_flat)
    return out.reshape(B, NA, NB)


# speedup vs baseline: 23.4126x; 1.0877x over previous
import functools

import jax
import jax.numpy as jnp
from jax import lax
from jax.experimental import pallas as pl
from jax.experimental.pallas import tpu as pltpu
from jax.experimental.pallas import tpu_sc as plsc

B = 8
NA = 128
NB = 256
L = 16
NVREG = NA // L
NEG = -1e20
LN2 = 0.6931471805599453


def _softlog13(s):
    hi = s >= 2.0
    y = jnp.where(hi, 0.5 * s, s)
    z = (y - 1.0) / (y + 1.0)
    z2 = z * z
    p = z2 * (1.0 / 9.0) + (1.0 / 7.0)
    p = p * z2 + (1.0 / 5.0)
    p = p * z2 + (1.0 / 3.0)
    p = p * z2 + 1.0
    return 2.0 * z * p + jnp.where(hi, LN2, 0.0)


def _lse3(a, b, c):
    m = jnp.maximum(jnp.maximum(a, b), c)
    s = jnp.exp(a - m) + jnp.exp(b - m) + jnp.exp(c - m)
    return m + _softlog13(s)


def _dg(x, idx):
    return x.at[idx].get(mode="promise_in_bounds")


def _dtw_body(w_hbm, out_hbm, w_v, out_v):
    wid = lax.axis_index("s") * 2 + lax.axis_index("c")

    @pl.when(wid < B)
    def _():
        pltpu.sync_copy(w_hbm.at[wid], w_v)

        lanes = lax.iota(jnp.int32, L)
        neg = jnp.full((L,), NEG, jnp.float32)
        lane0 = lanes == 0
        sh_idx = jnp.maximum(lanes - 1, 0)
        hi_idx = jnp.full((L,), L - 1, jnp.int32)

        ivecs = [lanes + (v * L + 1) for v in range(NVREG)]
        c255 = [iv * (NB - 1) for iv in ivecs]

        d1 = [neg] * NVREG
        s1 = [neg] * NVREG
        s2 = [jnp.where(lane0, 0.0, NEG) if v == 0 else neg
              for v in range(NVREG)]

        def make_step(lo_g, hi_g, mask_kind):
            def step(k, carry):
                d1 = list(carry[:NVREG])
                s1 = list(carry[NVREG:2 * NVREG])
                s2 = list(carry[2 * NVREG:])
                new, news = list(d1), list(s1)
                for v in range(lo_g, hi_g):
                    wofs = c255[v] + (k - (NB + 1))
                    w = plsc.load_gather(w_v, [wofs])
                    val = _lse3(d1[v], s1[v], s2[v]) + w
                    if mask_kind == "grow" and v == hi_g - 1:
                        msk = ivecs[v] <= k - 1
                    elif mask_kind == "shrink" and v == lo_g:
                        msk = ivecs[v] >= k - NB
                    else:
                        msk = None
                    if msk is None:
                        plsc.store_scatter(out_v, [wofs], val)
                    else:
                        val = jnp.where(msk, val, NEG)
                        plsc.store_scatter(out_v, [wofs], val, mask=msk)
                    carrier = neg if v == lo_g else _dg(new[v - 1], hi_idx)
                    news[v] = jnp.where(lane0, carrier, _dg(val, sh_idx))
                    new[v] = val
                if hi_g < NVREG:
                    news[hi_g] = jnp.where(
                        lane0, _dg(new[hi_g - 1], hi_idx), s1[hi_g])
                return tuple(new) + tuple(news) + tuple(s1)
            return step

        carry = tuple(d1) + tuple(s1) + tuple(s2)
        for g in range(1, NVREG + 1):
            carry = lax.fori_loop(L * (g - 1) + 2, L * g + 2,
                                  make_step(0, g, "grow"), carry)
        carry = lax.fori_loop(NA + 2, NB + 2,
                              make_step(0, NVREG, "full"), carry)
        for h in range(NVREG):
            carry = lax.fori_loop(NB + 2 + L * h,
                                  min(NB + 2 + L * (h + 1), NA + NB + 1),
                                  make_step(h, NVREG, "shrink"), carry)
        pltpu.sync_copy(out_v, out_hbm.at[wid])


@jax.jit
def kernel(W):
    w_flat = W.reshape(B, NA * NB)
    mesh = plsc.VectorSubcoreMesh(core_axis_name="c", subcore_axis_name="s")
    out = pl.kernel(
        _dtw_body,
        mesh=mesh,
        compiler_params=pltpu.CompilerParams(needs_layout_passes=False),
        out_type=jax.ShapeDtypeStruct((B, NA * NB), jnp.float32),
        scratch_types=[
            pltpu.VMEM((NA * NB,), jnp.float32),
            pltpu.VMEM((NA * NB,), jnp.float32),
        ],
    )(w_flat)
    return out.reshape(B, NA, NB)


# speedup vs baseline: 30.6061x; 1.3073x over previous
import functools

import jax
import jax.numpy as jnp
from jax import lax
from jax.experimental import pallas as pl
from jax.experimental.pallas import tpu as pltpu
from jax.experimental.pallas import tpu_sc as plsc

B = 8
NA = 128
NB = 256
L = 16
NVREG = NA // L
NEG = -1e20
LN2 = 0.6931471805599453


_LOGC = (-2.168550998512674, 4.574143954790401, -4.4891742371654155,
         3.2933146977645094, -1.6673165459822807, 0.5658843418242644,
         -0.12286688927625052, 0.015418058541798901, -0.0008502949466361012)


def _softlog13(s):
    c = _LOGC
    s2 = s * s
    s4 = s2 * s2
    q0 = c[1] * s + c[0]
    q1 = c[3] * s + c[2]
    q2 = c[5] * s + c[4]
    q3 = c[7] * s + c[6]
    r0 = q1 * s2 + q0
    r1 = q3 * s2 + q2
    return (c[8] * s4 + r1) * s4 + r0


def _lse3(a, b, c, mw):
    m = jnp.maximum(jnp.maximum(a, b), c)
    s = (jnp.exp(a - m) + jnp.exp(b - m)) + jnp.exp(c - m)
    return _softlog13(s) + (m + mw)


def _dg(x, idx):
    return x.at[idx].get(mode="promise_in_bounds")


def _dtw_body(w_hbm, out_hbm, w_v, out_v):
    wid = lax.axis_index("s") * 2 + lax.axis_index("c")

    @pl.when(wid < B)
    def _():
        pltpu.sync_copy(w_hbm.at[wid], w_v)

        lanes = lax.iota(jnp.int32, L)
        neg = jnp.full((L,), NEG, jnp.float32)
        lane0 = lanes == 0
        sh_idx = jnp.maximum(lanes - 1, 0)
        hi_idx = jnp.full((L,), L - 1, jnp.int32)

        ivecs = [lanes + (v * L + 1) for v in range(NVREG)]
        c255 = [iv * (NB - 1) for iv in ivecs]

        d1 = [neg] * NVREG
        s1 = [neg] * NVREG
        s2 = [jnp.where(lane0, 0.0, NEG) if v == 0 else neg
              for v in range(NVREG)]

        def make_step(lo_g, hi_g, mask_kind):
            def step(k, carry):
                d1 = list(carry[:NVREG])
                s1 = list(carry[NVREG:2 * NVREG])
                s2 = list(carry[2 * NVREG:])
                new, news = list(d1), list(s1)
                for v in range(lo_g, hi_g):
                    wofs = c255[v] + (k - (NB + 1))
                    w = plsc.load_gather(w_v, [wofs])
                    val = _lse3(d1[v], s1[v], s2[v], w)
                    if mask_kind == "grow" and v == hi_g - 1:
                        msk = ivecs[v] <= k - 1
                    elif mask_kind == "shrink" and v == lo_g:
                        msk = ivecs[v] >= k - NB
                    else:
                        msk = None
                    if msk is None:
                        plsc.store_scatter(out_v, [wofs], val)
                    else:
                        val = jnp.where(msk, val, NEG)
                        plsc.store_scatter(out_v, [wofs], val, mask=msk)
                    carrier = neg if v == lo_g else _dg(new[v - 1], hi_idx)
                    news[v] = jnp.where(lane0, carrier, _dg(val, sh_idx))
                    new[v] = val
                if hi_g < NVREG:
                    news[hi_g] = jnp.where(
                        lane0, _dg(new[hi_g - 1], hi_idx), s1[hi_g])
                return tuple(new) + tuple(news) + tuple(s1)
            return step

        carry = tuple(d1) + tuple(s1) + tuple(s2)
        for g in range(1, NVREG + 1):
            carry = lax.fori_loop(L * (g - 1) + 2, L * g + 2,
                                  make_step(0, g, "grow"), carry)
        carry = lax.fori_loop(NA + 2, NB + 2,
                              make_step(0, NVREG, "full"), carry)
        for h in range(NVREG):
            carry = lax.fori_loop(NB + 2 + L * h,
                                  min(NB + 2 + L * (h + 1), NA + NB + 1),
                                  make_step(h, NVREG, "shrink"), carry)
        pltpu.sync_copy(out_v, out_hbm.at[wid])


@jax.jit
def kernel(W):
    w_flat = W.reshape(B, NA * NB)
    mesh = plsc.VectorSubcoreMesh(core_axis_name="c", subcore_axis_name="s")
    out = pl.kernel(
        _dtw_body,
        mesh=mesh,
        compiler_params=pltpu.CompilerParams(needs_layout_passes=False),
        out_type=jax.ShapeDtypeStruct((B, NA * NB), jnp.float32),
        scratch_types=[
            pltpu.VMEM((NA * NB,), jnp.float32),
            pltpu.VMEM((NA * NB,), jnp.float32),
        ],
    )(w_flat)
    return out.reshape(B, NA, NB)


# speedup vs baseline: 31.3119x; 1.0231x over previous
import functools

import jax
import jax.numpy as jnp
from jax import lax
from jax.experimental import pallas as pl
from jax.experimental.pallas import tpu as pltpu
from jax.experimental.pallas import tpu_sc as plsc

B = 8
NA = 128
NB = 256
L = 16
NVREG = NA // L
NEG = -1e20
LN2 = 0.6931471805599453


_LOGC = (-1.7340271472930908, 2.8420932292938232, -1.5665243864059448,
         0.5587959289550781, -0.10901706665754318, 0.008839796297252178)


def _softlog13(s):
    c = _LOGC
    p = c[5] * s + c[4]
    p = p * s + c[3]
    p = p * s + c[2]
    p = p * s + c[1]
    return p * s + c[0]


def _lse3(a, b, c, mw):
    m = jnp.maximum(jnp.maximum(a, b), c)
    s = (jnp.exp(a - m) + jnp.exp(b - m)) + jnp.exp(c - m)
    return _softlog13(s) + (m + mw)


def _dg(x, idx):
    return x.at[idx].get(mode="promise_in_bounds")


def _dtw_body(w_hbm, out_hbm, w_v, out_v):
    wid = lax.axis_index("s") * 2 + lax.axis_index("c")

    @pl.when(wid < B)
    def _():
        pltpu.sync_copy(w_hbm.at[wid], w_v)

        lanes = lax.iota(jnp.int32, L)
        neg = jnp.full((L,), NEG, jnp.float32)
        lane0 = lanes == 0
        sh_idx = jnp.maximum(lanes - 1, 0)
        hi_idx = jnp.full((L,), L - 1, jnp.int32)

        ivecs = [lanes + (v * L + 1) for v in range(NVREG)]
        c255 = [iv * (NB - 1) for iv in ivecs]

        d1 = [neg] * NVREG
        s1 = [neg] * NVREG
        s2 = [jnp.where(lane0, 0.0, NEG) if v == 0 else neg
              for v in range(NVREG)]

        def make_step(lo_g, hi_g, mask_kind):
            def step(k, carry):
                d1 = list(carry[:NVREG])
                s1 = list(carry[NVREG:2 * NVREG])
                s2 = list(carry[2 * NVREG:])
                new, news = list(d1), list(s1)
                for v in range(lo_g, hi_g):
                    wofs = c255[v] + (k - (NB + 1))
                    w = plsc.load_gather(w_v, [wofs])
                    val = _lse3(d1[v], s1[v], s2[v], w)
                    if mask_kind == "grow" and v == hi_g - 1:
                        msk = ivecs[v] <= k - 1
                    elif mask_kind == "shrink" and v == lo_g:
                        msk = ivecs[v] >= k - NB
                    else:
                        msk = None
                    if msk is None:
                        plsc.store_scatter(out_v, [wofs], val)
                    else:
                        val = jnp.where(msk, val, NEG)
                        plsc.store_scatter(out_v, [wofs], val, mask=msk)
                    carrier = neg if v == lo_g else _dg(new[v - 1], hi_idx)
                    news[v] = jnp.where(lane0, carrier, _dg(val, sh_idx))
                    new[v] = val
                if hi_g < NVREG:
                    news[hi_g] = jnp.where(
                        lane0, _dg(new[hi_g - 1], hi_idx), s1[hi_g])
                return tuple(new) + tuple(news) + tuple(s1)
            return step

        carry = tuple(d1) + tuple(s1) + tuple(s2)
        for g in range(1, NVREG + 1):
            carry = lax.fori_loop(L * (g - 1) + 2, L * g + 2,
                                  make_step(0, g, "grow"), carry)
        carry = lax.fori_loop(NA + 2, NB + 2,
                              make_step(0, NVREG, "full"), carry)
        for h in range(NVREG):
            carry = lax.fori_loop(NB + 2 + L * h,
                                  min(NB + 2 + L * (h + 1), NA + NB + 1),
                                  make_step(h, NVREG, "shrink"), carry)
        pltpu.sync_copy(out_v, out_hbm.at[wid])


@jax.jit
def kernel(W):
    w_flat = W.reshape(B, NA * NB)
    mesh = plsc.VectorSubcoreMesh(core_axis_name="c", subcore_axis_name="s")
    out = pl.kernel(
        _dtw_body,
        mesh=mesh,
        compiler_params=pltpu.CompilerParams(needs_layout_passes=False),
        out_type=jax.ShapeDtypeStruct((B, NA * NB), jnp.float32),
        scratch_types=[
            pltpu.VMEM((NA * NB,), jnp.float32),
            pltpu.VMEM((NA * NB,), jnp.float32),
        ],
    )(w_flat)
    return out.reshape(B, NA, NB)


# speedup vs baseline: 32.3151x; 1.0320x over previous
import functools

import jax
import jax.numpy as jnp
from jax import lax
from jax.experimental import pallas as pl
from jax.experimental.pallas import tpu as pltpu
from jax.experimental.pallas import tpu_sc as plsc

B = 8
NA = 128
NB = 256
L = 16
NVREG = NA // L
NEG = -1e20
LN2 = 0.6931471805599453


_LOGC = (-1.270023625799189, 1.6696577339751741,
         -0.4448658805190204, 0.050633374560152114)


def _softlog13(s):
    c = _LOGC
    p = c[3] * s + c[2]
    p = p * s + c[1]
    return p * s + c[0]


def _lse3(a, b, c, mw):
    m = jnp.maximum(jnp.maximum(a, b), c)
    s = (jnp.exp(a - m) + jnp.exp(b - m)) + jnp.exp(c - m)
    return _softlog13(s) + (m + mw)


def _dg(x, idx):
    return x.at[idx].get(mode="promise_in_bounds")


def _dtw_body(w_hbm, out_hbm, w_v, out_v):
    wid = lax.axis_index("s") * 2 + lax.axis_index("c")

    @pl.when(wid < B)
    def _():
        pltpu.sync_copy(w_hbm.at[wid], w_v)

        lanes = lax.iota(jnp.int32, L)
        neg = jnp.full((L,), NEG, jnp.float32)
        lane0 = lanes == 0
        sh_idx = jnp.maximum(lanes - 1, 0)
        hi_idx = jnp.full((L,), L - 1, jnp.int32)

        ivecs = [lanes + (v * L + 1) for v in range(NVREG)]
        c255 = [iv * (NB - 1) for iv in ivecs]

        d1 = [neg] * NVREG
        s1 = [neg] * NVREG
        s2 = [jnp.where(lane0, 0.0, NEG) if v == 0 else neg
              for v in range(NVREG)]

        def make_step(lo_g, hi_g, mask_kind):
            def step(k, carry):
                d1 = list(carry[:NVREG])
                s1 = list(carry[NVREG:2 * NVREG])
                s2 = list(carry[2 * NVREG:])
                new, news = list(d1), list(s1)
                for v in range(lo_g, hi_g):
                    wofs = c255[v] + (k - (NB + 1))
                    w = plsc.load_gather(w_v, [wofs])
                    val = _lse3(d1[v], s1[v], s2[v], w)
                    if mask_kind == "grow" and v == hi_g - 1:
                        msk = ivecs[v] <= k - 1
                    elif mask_kind == "shrink" and v == lo_g:
                        msk = ivecs[v] >= k - NB
                    else:
                        msk = None
                    if msk is None:
                        plsc.store_scatter(out_v, [wofs], val)
                    else:
                        val = jnp.where(msk, val, NEG)
                        plsc.store_scatter(out_v, [wofs], val, mask=msk)
                    carrier = neg if v == lo_g else _dg(new[v - 1], hi_idx)
                    news[v] = jnp.where(lane0, carrier, _dg(val, sh_idx))
                    new[v] = val
                if hi_g < NVREG:
                    news[hi_g] = jnp.where(
                        lane0, _dg(new[hi_g - 1], hi_idx), s1[hi_g])
                return tuple(new) + tuple(news) + tuple(s1)
            return step

        carry = tuple(d1) + tuple(s1) + tuple(s2)
        for g in range(1, NVREG + 1):
            carry = lax.fori_loop(L * (g - 1) + 2, L * g + 2,
                                  make_step(0, g, "grow"), carry)
        carry = lax.fori_loop(NA + 2, NB + 2,
                              make_step(0, NVREG, "full"), carry)
        for h in range(NVREG):
            carry = lax.fori_loop(NB + 2 + L * h,
                                  min(NB + 2 + L * (h + 1), NA + NB + 1),
                                  make_step(h, NVREG, "shrink"), carry)
        pltpu.sync_copy(out_v, out_hbm.at[wid])


@jax.jit
def kernel(W):
    w_flat = W.reshape(B, NA * NB)
    mesh = plsc.VectorSubcoreMesh(core_axis_name="c", subcore_axis_name="s")
    out = pl.kernel(
        _dtw_body,
        mesh=mesh,
        compiler_params=pltpu.CompilerParams(needs_layout_passes=False),
        out_type=jax.ShapeDtypeStruct((B, NA * NB), jnp.float32),
        scratch_types=[
            pltpu.VMEM((NA * NB,), jnp.float32),
            pltpu.VMEM((NA * NB,), jnp.float32),
        ],
    )(w_flat)
    return out.reshape(B, NA, NB)
